# baseline (speedup 1.0000x reference)
import jax
import jax.numpy as jnp
from jax.experimental import pallas as pl
from jax.experimental.pallas import tpu as pltpu

CH = 32
KSZ = 5
FC1_DIM = 32
LEAKY_ADJ = 0.01
LEAKY_GCONV = 0.1


def _round_up(n, m):
    return ((n + m - 1) // m) * m


def _conv_out_len(l):
    return (l + 2 * 2 - KSZ) // 2 + 1


def _cnn_lens(l):
    l1 = _conv_out_len(l)
    l2 = _conv_out_len(l1)
    l3 = _conv_out_len(l2)
    return l1, l2, l3


def _cnn_s_offsets(l):
    l1, l2, _ = _cnn_lens(l)
    o1 = 0
    o2 = _round_up(o1 + KSZ * l1, 8)
    o3 = _round_up(o2 + KSZ * l2, 8)
    return o1, o2, o3


def _build_cnn_mats(S, CW, CB, WF1, WF2, L):
    l1, l2, l3 = _cnn_lens(L)
    o1, o2, o3 = _cnn_s_offsets(L)
    S1 = S[o1:o1 + KSZ * l1, :L].reshape(KSZ, l1, L)
    S2 = S[o2:o2 + KSZ * l2, :l1].reshape(KSZ, l2, l1)
    S3 = S[o3:o3 + KSZ * l3, :l2].reshape(KSZ, l3, l2)
    w1 = CW[:, 0:1, :]
    w2 = CW[:, 8:8 + CH, :]
    w3 = CW[:, 8 + CH:8 + 2 * CH, :]
    A1T = jnp.einsum('tc,toi->coi', w1[:, 0, :], S1).reshape(CH * l1, L).T
    A2T = jnp.einsum('tic,top->coip', w2, S2).reshape(CH * l2, CH * l1).T
    A3T = jnp.einsum('tic,top->coip', w3, S3).reshape(CH * l3, CH * l2).T
    F1 = WF1.transpose(1, 0, 2).reshape(CH * l3, FC1_DIM)
    feat = WF2.shape[1]
    b1 = jnp.repeat(CB[0, :], l1)[None, :]
    b2 = jnp.repeat(CB[1, :], l2)[None, :]
    b3 = jnp.repeat(CB[2, :], l3)[None, :]
    bf1 = CB[3:4, :FC1_DIM]
    bf2 = CB[4:5, :feat]
    return A1T, A2T, A3T, F1, WF2, b1, b2, b3, bf1, bf2


def _bdot(a, b):
    return jnp.dot(a.astype(jnp.bfloat16), b.astype(jnp.bfloat16),
                   preferred_element_type=jnp.float32)


def _cnn_kernel(x_ref, a1_ref, a2_ref, a3_ref, f1_ref, f2_ref,
                b1_ref, b2_ref, b3_ref, bf1_ref, bf2_ref, o_ref):
    z = _bdot(x_ref[...], a1_ref[...])
    z = jnp.maximum(z + b1_ref[...], 0.0)
    z = _bdot(z, a2_ref[...])
    z = jnp.maximum(z + b2_ref[...], 0.0)
    z = _bdot(z, a3_ref[...])
    z = jnp.maximum(z + b3_ref[...], 0.0)
    z = _bdot(z, f1_ref[...])
    z = jnp.maximum(z + bf1_ref[...], 0.0)
    o_ref[...] = _bdot(z, f2_ref[...]) + bf2_ref[...]


def _cnn_forward(S, CW, CB, WF1, WF2, xall):
    N, L = xall.shape
    mats = _build_cnn_mats(S, CW, CB, WF1, WF2, L)
    mats = tuple(m.astype(jnp.bfloat16) for m in mats[:5]) + mats[5:]
    feat = WF2.shape[1]
    NB = 512
    npad = _round_up(N, NB)
    xall = xall.astype(jnp.bfloat16)
    if npad != N:
        xall = jnp.pad(xall, ((0, npad - N), (0, 0)))
    grid = npad // NB
    full = lambda a: pl.BlockSpec(a.shape, lambda i: (0,) * a.ndim)
    out = pl.pallas_call(
        _cnn_kernel,
        out_shape=jax.ShapeDtypeStruct((npad, feat), jnp.float32),
        grid=(grid,),
        in_specs=[pl.BlockSpec((NB, L), lambda i: (i, 0))] +
                 [full(m) for m in mats],
        out_specs=pl.BlockSpec((NB, feat), lambda i: (i, 0)),
        compiler_params=pltpu.CompilerParams(
            dimension_semantics=("parallel",)),
    )(xall, *mats)
    return out[:N]


def _gnn_kernel(nodes_ref, gjb_ref, gtb_ref, jm_ref, bd_ref,
                w1g_ref, c2w_ref, c3t_ref, b_ref,
                logp_ref, a_ref, xsc):
    NE = nodes_ref.shape[0]
    V = nodes_ref.shape[1]
    D0 = nodes_ref.shape[2]
    DM = w1g_ref.shape[1]
    num_layers = c2w_ref.shape[0] - 1
    nway = logp_ref.shape[2]
    c1out = c2w_ref.shape[1]
    c2out = c2w_ref.shape[2]
    gout = w1g_ref.shape[2] - c1out
    gb_off = c1out + c2out
    c3b_off = gb_off + gout

    xsc[...] = jnp.zeros(xsc.shape, jnp.float32)
    xsc[:, 0:D0] = nodes_ref[...].reshape(NE * V, D0)

    for l in range(num_layers + 1):
        x = xsc[...]
        phi = jnp.abs(_bdot(gjb_ref[...], x))
        h = _bdot(phi, w1g_ref[l, :, 0:c1out]) + b_ref[l:l + 1, 0:c1out]
        h = jnp.where(h >= 0, h, h * LEAKY_ADJ)
        h = _bdot(h, c2w_ref[l]) + b_ref[l:l + 1, c1out:gb_off]
        h = jnp.where(h >= 0, h, h * LEAKY_ADJ)
        s = _bdot(h, c3t_ref[l]) + b_ref[l:l + 1, c3b_off:c3b_off + 1]
        smat = _bdot(gtb_ref[...], s * jm_ref[...])
        m = jnp.max(smat, axis=1, keepdims=True)
        e = jnp.exp(smat - m)
        adj = e / jnp.sum(e, axis=1, keepdims=True)
        adjb = jnp.tile(adj, (1, NE)) * bd_ref[...]
        xa = _bdot(adjb, x)
        if l < num_layers:
            y = _bdot(xa, w1g_ref[l, :, c1out:c1out + gout])
            y = y + b_ref[l:l + 1, gb_off:gb_off + gout]
            y = jnp.where(y >= 0, y, y * LEAKY_GCONV)
            xsc[:, D0 + l * gout:D0 + (l + 1) * gout] = y
        else:
            a_ref[...] = adj.reshape(NE, V, V)
            x0 = xa.reshape(NE, V, DM)[:, 0, :]
            logits = _bdot(x0, w1g_ref[l, :, c1out:c1out + nway])
            logits = logits + b_ref[l:l + 1, gb_off:gb_off + nway]
            mm = jnp.max(logits, axis=-1, keepdims=True)
            lse = jnp.log(jnp.sum(jnp.exp(logits - mm), axis=-1, keepdims=True))
            logp_ref[...] = (logits - mm - lse).reshape(NE, 1, nway)


def _gnn_forward(GJJ, GT, W1G, C2W, C3W, BIAS, nodes, nway):
    B, V, D0 = nodes.shape
    DM = W1G.shape[1]
    W1G = W1G.astype(jnp.bfloat16)
    C2W = C2W.astype(jnp.bfloat16)
    C3t = C3W.astype(jnp.bfloat16)
    NE = 16
    while B % NE:
        NE //= 2
    eye = jnp.eye(NE, dtype=jnp.float32)
    gj = GJJ[:, 0:V]
    jm = GJJ[:, V:2 * V]
    gjb = jnp.kron(eye, gj).astype(jnp.bfloat16)
    gtb = jnp.kron(eye, GT).astype(jnp.bfloat16)
    jmt = jnp.tile(jm, (NE, 1))
    bd = jnp.kron(eye, jnp.ones((V, V), jnp.float32))
    full = lambda a: pl.BlockSpec(a.shape, lambda i: (0,) * a.ndim)
    logp, a = pl.pallas_call(
        _gnn_kernel,
        out_shape=(jax.ShapeDtypeStruct((B, 1, nway), jnp.float32),
                   jax.ShapeDtypeStruct((B, V, V), jnp.float32)),
        grid=(B // NE,),
        in_specs=[pl.BlockSpec((NE, V, D0), lambda i: (i, 0, 0)),
                  full(gjb), full(gtb), full(jmt), full(bd),
                  full(W1G), full(C2W), full(C3t), full(BIAS)],
        out_specs=(pl.BlockSpec((NE, 1, nway), lambda i: (i, 0, 0)),
                   pl.BlockSpec((NE, V, V), lambda i: (i, 0, 0))),
        scratch_shapes=[pltpu.VMEM((NE * V, DM), jnp.float32)],
        compiler_params=pltpu.CompilerParams(
            dimension_semantics=("parallel",)),
    )(nodes, gjb, gtb, jmt, bd, W1G, C2W, C3t, BIAS)
    return logp[:, 0, :], a


def kernel(cnn_S, cnn_CW, cnn_CB, cnn_WF1, cnn_WF2,
           gnn_GJJ, gnn_GT, gnn_W1G, gnn_C2W, gnn_C3W, gnn_BIAS,
           x, xi, one_hot_yi):
    B, _, L = x.shape
    M = xi.shape[1]
    V = M + 1
    nway = one_hot_yi.shape[2]
    all_inp = jnp.concatenate([x[:, None], xi], axis=1).reshape(B * V, L)
    feats = _cnn_forward(cnn_S, cnn_CW, cnn_CB, cnn_WF1, cnn_WF2, all_inp)
    feats = feats.reshape(B, V, -1)
    uniform_pad = jnp.full((B, 1, nway), 1.0 / nway, jnp.float32)
    labels = jnp.concatenate([uniform_pad, one_hot_yi.astype(jnp.float32)],
                             axis=1)
    nodes = jnp.concatenate([feats, labels], axis=2)
    return _gnn_forward(gnn_GJJ, gnn_GT, gnn_W1G, gnn_C2W, gnn_C3W,
                        gnn_BIAS, nodes, nway)


# speedup vs baseline: 1.5387x; 1.5387x over previous
import jax
import jax.numpy as jnp
from jax.experimental import pallas as pl
from jax.experimental.pallas import tpu as pltpu

CH = 32
KSZ = 5
FC1_DIM = 32
LEAKY_ADJ = 0.01
LEAKY_GCONV = 0.1


def _round_up(n, m):
    return ((n + m - 1) // m) * m


def _conv_out_len(l):
    return (l + 2 * 2 - KSZ) // 2 + 1


def _cnn_lens(l):
    l1 = _conv_out_len(l)
    l2 = _conv_out_len(l1)
    l3 = _conv_out_len(l2)
    return l1, l2, l3


def _cnn_s_offsets(l):
    l1, l2, _ = _cnn_lens(l)
    o1 = 0
    o2 = _round_up(o1 + KSZ * l1, 8)
    o3 = _round_up(o2 + KSZ * l2, 8)
    return o1, o2, o3


def _build_cnn_mats(S, CW, CB, WF1, WF2, L):
    l1, l2, l3 = _cnn_lens(L)
    o1, o2, o3 = _cnn_s_offsets(L)
    S1 = S[o1:o1 + KSZ * l1, :L].reshape(KSZ, l1, L)
    S2 = S[o2:o2 + KSZ * l2, :l1].reshape(KSZ, l2, l1)
    S3 = S[o3:o3 + KSZ * l3, :l2].reshape(KSZ, l3, l2)
    w1 = CW[:, 0:1, :]
    w2 = CW[:, 8:8 + CH, :]
    w3 = CW[:, 8 + CH:8 + 2 * CH, :]
    A1T = jnp.einsum('tc,toi->coi', w1[:, 0, :], S1).reshape(CH * l1, L).T
    A2T = jnp.einsum('tic,top->coip', w2, S2).reshape(CH * l2, CH * l1).T
    A3T = jnp.einsum('tic,top->coip', w3, S3).reshape(CH * l3, CH * l2).T
    F1 = WF1.transpose(1, 0, 2).reshape(CH * l3, FC1_DIM)
    feat = WF2.shape[1]
    b1 = jnp.repeat(CB[0, :], l1)[None, :]
    b2 = jnp.repeat(CB[1, :], l2)[None, :]
    b3 = jnp.repeat(CB[2, :], l3)[None, :]
    bf1 = CB[3:4, :FC1_DIM]
    bf2 = CB[4:5, :feat]
    return A1T, A2T, A3T, F1, WF2, b1, b2, b3, bf1, bf2


def _bdot(a, b):
    return jnp.dot(a.astype(jnp.bfloat16), b.astype(jnp.bfloat16),
                   preferred_element_type=jnp.float32)


def _cnn_kernel(x_ref, a1_ref, a2_ref, a3_ref, f1_ref, f2_ref,
                b1_ref, b2_ref, b3_ref, bf1_ref, bf2_ref, o_ref):
    z = _bdot(x_ref[...], a1_ref[...])
    z = jnp.maximum(z + b1_ref[...], 0.0)
    z = _bdot(z, a2_ref[...])
    z = jnp.maximum(z + b2_ref[...], 0.0)
    z = _bdot(z, a3_ref[...])
    z = jnp.maximum(z + b3_ref[...], 0.0)
    z = _bdot(z, f1_ref[...])
    z = jnp.maximum(z + bf1_ref[...], 0.0)
    o_ref[...] = _bdot(z, f2_ref[...]) + bf2_ref[...]


def _cnn_forward(S, CW, CB, WF1, WF2, xall):
    N, L = xall.shape
    mats = _build_cnn_mats(S, CW, CB, WF1, WF2, L)
    mats = tuple(m.astype(jnp.bfloat16) for m in mats[:5]) + mats[5:]
    feat = WF2.shape[1]
    NB = 512
    npad = _round_up(N, NB)
    xall = xall.astype(jnp.bfloat16)
    if npad != N:
        xall = jnp.pad(xall, ((0, npad - N), (0, 0)))
    grid = npad // NB
    full = lambda a: pl.BlockSpec(a.shape, lambda i: (0,) * a.ndim)
    out = pl.pallas_call(
        _cnn_kernel,
        out_shape=jax.ShapeDtypeStruct((npad, feat), jnp.float32),
        grid=(grid,),
        in_specs=[pl.BlockSpec((NB, L), lambda i: (i, 0))] +
                 [full(m) for m in mats],
        out_specs=pl.BlockSpec((NB, feat), lambda i: (i, 0)),
        compiler_params=pltpu.CompilerParams(
            dimension_semantics=("parallel",)),
    )(xall, *mats)
    return out[:N]


def _gnn_kernel(nodes_ref, gjb_ref, gtb_ref, jm_ref, bd_ref,
                w1g_ref, c2w_ref, c3t_ref, b_ref,
                logp_ref, a_ref, xsc):
    NE = nodes_ref.shape[0]
    V = nodes_ref.shape[1]
    D0 = nodes_ref.shape[2]
    DM = w1g_ref.shape[1]
    num_layers = c2w_ref.shape[0] - 1
    nway = logp_ref.shape[2]
    c1out = c2w_ref.shape[1]
    c2out = c2w_ref.shape[2]
    gout = w1g_ref.shape[2] - c1out
    gb_off = c1out + c2out
    c3b_off = gb_off + gout

    xsc[...] = jnp.zeros(xsc.shape, jnp.float32)
    xsc[:, 0:D0] = nodes_ref[...].reshape(NE * V, D0)

    for l in range(num_layers + 1):
        x = xsc[...]
        phi = jnp.abs(_bdot(gjb_ref[...], x))
        h = _bdot(phi, w1g_ref[l, :, 0:c1out]) + b_ref[l:l + 1, 0:c1out]
        h = jnp.where(h >= 0, h, h * LEAKY_ADJ)
        h = _bdot(h, c2w_ref[l]) + b_ref[l:l + 1, c1out:gb_off]
        h = jnp.where(h >= 0, h, h * LEAKY_ADJ)
        s = _bdot(h, c3t_ref[l]) + b_ref[l:l + 1, c3b_off:c3b_off + 1]
        smat = _bdot(gtb_ref[...], s * jm_ref[...])
        m = jnp.max(smat, axis=1, keepdims=True)
        e = jnp.exp(smat - m)
        adj = e / jnp.sum(e, axis=1, keepdims=True)
        adjb = jnp.tile(adj, (1, NE)) * bd_ref[...]
        xa = _bdot(adjb, x)
        if l < num_layers:
            y = _bdot(xa, w1g_ref[l, :, c1out:c1out + gout])
            y = y + b_ref[l:l + 1, gb_off:gb_off + gout]
            y = jnp.where(y >= 0, y, y * LEAKY_GCONV)
            xsc[:, D0 + l * gout:D0 + (l + 1) * gout] = y
        else:
            a_ref[...] = adj.reshape(NE, V, V)
            x0 = xa.reshape(NE, V, DM)[:, 0, :]
            logits = _bdot(x0, w1g_ref[l, :, c1out:c1out + nway])
            logits = logits + b_ref[l:l + 1, gb_off:gb_off + nway]
            mm = jnp.max(logits, axis=-1, keepdims=True)
            lse = jnp.log(jnp.sum(jnp.exp(logits - mm), axis=-1, keepdims=True))
            logp_ref[...] = (logits - mm - lse).reshape(NE, 1, nway)


def _gnn_forward(GJJ, GT, W1G, C2W, C3W, BIAS, nodes, nway):
    B, V, D0 = nodes.shape
    DM = W1G.shape[1]
    W1G = W1G.astype(jnp.bfloat16)
    C2W = C2W.astype(jnp.bfloat16)
    C3t = C3W.astype(jnp.bfloat16)
    NE = 8
    while B % NE:
        NE //= 2
    eye = jnp.eye(NE, dtype=jnp.float32)
    gj = GJJ[:, 0:V]
    jm = GJJ[:, V:2 * V]
    gjb = jnp.kron(eye, gj).astype(jnp.bfloat16)
    gtb = jnp.kron(eye, GT).astype(jnp.bfloat16)
    jmt = jnp.tile(jm, (NE, 1))
    bd = jnp.kron(eye, jnp.ones((V, V), jnp.float32))
    full = lambda a: pl.BlockSpec(a.shape, lambda i: (0,) * a.ndim)
    logp, a = pl.pallas_call(
        _gnn_kernel,
        out_shape=(jax.ShapeDtypeStruct((B, 1, nway), jnp.float32),
                   jax.ShapeDtypeStruct((B, V, V), jnp.float32)),
        grid=(B // NE,),
        in_specs=[pl.BlockSpec((NE, V, D0), lambda i: (i, 0, 0)),
                  full(gjb), full(gtb), full(jmt), full(bd),
                  full(W1G), full(C2W), full(C3t), full(BIAS)],
        out_specs=(pl.BlockSpec((NE, 1, nway), lambda i: (i, 0, 0)),
                   pl.BlockSpec((NE, V, V), lambda i: (i, 0, 0))),
        scratch_shapes=[pltpu.VMEM((NE * V, DM), jnp.float32)],
        compiler_params=pltpu.CompilerParams(
            dimension_semantics=("parallel",)),
    )(nodes, gjb, gtb, jmt, bd, W1G, C2W, C3t, BIAS)
    return logp[:, 0, :], a


def kernel(cnn_S, cnn_CW, cnn_CB, cnn_WF1, cnn_WF2,
           gnn_GJJ, gnn_GT, gnn_W1G, gnn_C2W, gnn_C3W, gnn_BIAS,
           x, xi, one_hot_yi):
    B, _, L = x.shape
    M = xi.shape[1]
    V = M + 1
    nway = one_hot_yi.shape[2]
    all_inp = jnp.concatenate([x[:, None], xi], axis=1).reshape(B * V, L)
    feats = _cnn_forward(cnn_S, cnn_CW, cnn_CB, cnn_WF1, cnn_WF2, all_inp)
    feats = feats.reshape(B, V, -1)
    uniform_pad = jnp.full((B, 1, nway), 1.0 / nway, jnp.float32)
    labels = jnp.concatenate([uniform_pad, one_hot_yi.astype(jnp.float32)],
                             axis=1)
    nodes = jnp.concatenate([feats, labels], axis=2)
    return _gnn_forward(gnn_GJJ, gnn_GT, gnn_W1G, gnn_C2W, gnn_C3W,
                        gnn_BIAS, nodes, nway)
